# Initial kernel scaffold; baseline (speedup 1.0000x reference)
import jax
import jax.numpy as jnp
from jax.experimental import pallas as pl


def kernel(h, edge_index1, edge_index2, W_lin, b_lin, W_fc, b_fc, W_al, b_al, W_ar, b_ar):
    raise NotImplementedError("write your pallas kernel here")


# speedup vs baseline: 8.1260x; 8.1260x over previous
import jax
import jax.numpy as jnp
from jax import lax
from jax.experimental import pallas as pl
from jax.experimental.pallas import tpu as pltpu
from jax.experimental.pallas import tpu_sc as plsc

N_NODES = 10000
FEATS = 128
NUM_TILES = 16
NUM_CORES = 2
LANES = 128
N_PAD = 10240
ROWS_PER_TILE = N_PAD // NUM_TILES
CHUNKS = 160
EDGES_PER_TILE = CHUNKS * LANES
DEG_W = 16

import functools


@functools.lru_cache(maxsize=None)
def _sc_mesh():
    return plsc.VectorSubcoreMesh(
        core_axis_name="c", subcore_axis_name="s",
        num_cores=NUM_CORES, num_subcores=NUM_TILES,
    )


def _deg_body(dst_hbm, deg_hbm, dst_v, ones_v, stage_v, acc):
    c = lax.axis_index("c")
    s = lax.axis_index("s")
    wid = c * NUM_TILES + s
    pltpu.sync_copy(dst_hbm.at[pl.ds(wid * CHUNKS, CHUNKS)], dst_v)

    def zfill(i, carry):
        ones_v[i, :] = jnp.ones((DEG_W,), jnp.float32)
        stage_v[i, :] = jnp.zeros((DEG_W,), jnp.float32)
        return carry

    lax.fori_loop(0, LANES, zfill, 0)
    base = s * ROWS_PER_TILE
    for k in range(ROWS_PER_TILE // LANES):
        pltpu.sync_copy(stage_v, acc.at[pl.ds(base + k * LANES, LANES)])
    plsc.subcore_barrier()

    def body(j, carry):
        pltpu.sync_copy(ones_v, acc.at[dst_v.at[j]], add=True)
        return carry

    lax.fori_loop(0, CHUNKS, body, 0)
    plsc.subcore_barrier()
    for k in range(ROWS_PER_TILE // LANES):
        pltpu.sync_copy(acc.at[pl.ds(base + k * LANES, LANES)], stage_v)
        pltpu.sync_copy(
            stage_v, deg_hbm.at[pl.ds(c * N_PAD + base + k * LANES, LANES)]
        )


@functools.lru_cache(maxsize=None)
def _deg_call():
    return pl.kernel(
        _deg_body,
        out_type=jax.ShapeDtypeStruct((NUM_CORES * N_PAD, DEG_W), jnp.float32),
        mesh=_sc_mesh(),
        scratch_types=[
            pltpu.VMEM((CHUNKS, LANES), jnp.int32),
            pltpu.VMEM((LANES, DEG_W), jnp.float32),
            pltpu.VMEM((LANES, DEG_W), jnp.float32),
            pltpu.VMEM_SHARED((N_PAD, DEG_W), jnp.float32),
        ],
        compiler_params=pltpu.CompilerParams(use_tc_tiling_on_sc=False),
    )


HFEATS = FEATS // 2


def _prop_body(src_hbm, dst_hbm, ga_hbm, gb_hbm, outa_hbm, outb_hbm,
               src_v, dst_v, buf0, buf1, stage_v, zero_v, acc, sem0, sem1):
    c = lax.axis_index("c")
    s = lax.axis_index("s")
    wid = c * NUM_TILES + s
    pltpu.sync_copy(src_hbm.at[pl.ds(wid * CHUNKS, CHUNKS)], src_v)
    pltpu.sync_copy(dst_hbm.at[pl.ds(wid * CHUNKS, CHUNKS)], dst_v)

    def zfill(i, carry):
        for k in range(HFEATS // 16):
            zero_v[i, pl.ds(k * 16, 16)] = jnp.zeros((16,), jnp.float32)
        return carry

    lax.fori_loop(0, LANES, zfill, 0)
    base = s * ROWS_PER_TILE

    for f, (g_hbm, out_hbm) in enumerate(((ga_hbm, outa_hbm),
                                          (gb_hbm, outb_hbm))):
        for k in range(ROWS_PER_TILE // LANES):
            pltpu.sync_copy(zero_v, acc.at[pl.ds(base + k * LANES, LANES)])
        plsc.subcore_barrier()

        pltpu.async_copy(g_hbm.at[src_v.at[0]], buf0, sem0)

        def body(i, carry):
            j0 = 2 * i
            j1 = j0 + 1
            pltpu.async_copy(g_hbm.at[src_v.at[j1]], buf1, sem1)
            pltpu.make_async_copy(g_hbm.at[src_v.at[j0]], buf0, sem0).wait()
            pltpu.sync_copy(buf0, acc.at[dst_v.at[j0]], add=True)

            @pl.when(i < CHUNKS // 2 - 1)
            def _():
                pltpu.async_copy(g_hbm.at[src_v.at[j0 + 2]], buf0, sem0)

            pltpu.make_async_copy(g_hbm.at[src_v.at[j1]], buf1, sem1).wait()
            pltpu.sync_copy(buf1, acc.at[dst_v.at[j1]], add=True)
            return carry

        lax.fori_loop(0, CHUNKS // 2, body, 0)
        plsc.subcore_barrier()
        for k in range(ROWS_PER_TILE // LANES):
            pltpu.sync_copy(acc.at[pl.ds(base + k * LANES, LANES)], stage_v)
            pltpu.sync_copy(
                stage_v,
                out_hbm.at[pl.ds(c * N_PAD + base + k * LANES, LANES)],
            )


@functools.lru_cache(maxsize=None)
def _prop_call():
    return pl.kernel(
        _prop_body,
        out_type=(
            jax.ShapeDtypeStruct((NUM_CORES * N_PAD, HFEATS), jnp.float32),
            jax.ShapeDtypeStruct((NUM_CORES * N_PAD, HFEATS), jnp.float32),
        ),
        mesh=_sc_mesh(),
        scratch_types=[
            pltpu.VMEM((CHUNKS, LANES), jnp.int32),
            pltpu.VMEM((CHUNKS, LANES), jnp.int32),
            pltpu.VMEM((LANES, HFEATS), jnp.float32),
            pltpu.VMEM((LANES, HFEATS), jnp.float32),
            pltpu.VMEM((LANES, HFEATS), jnp.float32),
            pltpu.VMEM((LANES, HFEATS), jnp.float32),
            pltpu.VMEM_SHARED((N_PAD, HFEATS), jnp.float32),
            pltpu.SemaphoreType.DMA,
            pltpu.SemaphoreType.DMA,
        ],
        compiler_params=pltpu.CompilerParams(use_tc_tiling_on_sc=False),
    )


ROW_BLK = 1000
ROW_GRID = N_NODES // ROW_BLK


def _pre_body(h_ref, wlin_ref, blin_ref, wal_ref, bal_ref, deg1_ref, deg2_ref,
              hlin_ref, ga_ref, gb_ref, n1_ref, n2_ref, ai_ref):
    hlin = (
        jnp.dot(h_ref[...], wlin_ref[...].T, preferred_element_type=jnp.float32)
        + blin_ref[...][None, :]
    )
    hlin_ref[...] = hlin
    n1 = jnp.where(deg1_ref[...] > 0, lax.rsqrt(deg1_ref[...]), 0.0)
    n2 = jnp.where(deg2_ref[...] > 0, lax.rsqrt(deg2_ref[...]), 0.0)
    n1_ref[...] = n1
    n2_ref[...] = n2
    g1 = hlin * n1
    g2 = hlin * n2
    ga_ref[0] = g1[:, :HFEATS]
    ga_ref[1] = g2[:, :HFEATS]
    gb_ref[0] = g1[:, HFEATS:]
    gb_ref[1] = g2[:, HFEATS:]
    ai_ref[...] = jnp.sum(hlin * wal_ref[...], axis=1, keepdims=True) + bal_ref[0]


def _build_pre(interpret=False):
    return pl.pallas_call(
        _pre_body,
        interpret=interpret,
        grid=(ROW_GRID,),
    in_specs=[
        pl.BlockSpec((ROW_BLK, FEATS), lambda i: (i, 0)),
        pl.BlockSpec((FEATS, FEATS), lambda i: (0, 0)),
        pl.BlockSpec((FEATS,), lambda i: (0,)),
        pl.BlockSpec((1, FEATS), lambda i: (0, 0)),
        pl.BlockSpec(memory_space=pltpu.SMEM),
        pl.BlockSpec((ROW_BLK, 1), lambda i: (i, 0)),
        pl.BlockSpec((ROW_BLK, 1), lambda i: (i, 0)),
    ],
    out_specs=(
        pl.BlockSpec((ROW_BLK, FEATS), lambda i: (i, 0)),
        pl.BlockSpec((2, ROW_BLK, HFEATS), lambda i: (0, i, 0)),
        pl.BlockSpec((2, ROW_BLK, HFEATS), lambda i: (0, i, 0)),
        pl.BlockSpec((ROW_BLK, 1), lambda i: (i, 0)),
        pl.BlockSpec((ROW_BLK, 1), lambda i: (i, 0)),
        pl.BlockSpec((ROW_BLK, 1), lambda i: (i, 0)),
    ),
    out_shape=(
        jax.ShapeDtypeStruct((N_NODES, FEATS), jnp.float32),
        jax.ShapeDtypeStruct((2, N_NODES, HFEATS), jnp.float32),
        jax.ShapeDtypeStruct((2, N_NODES, HFEATS), jnp.float32),
        jax.ShapeDtypeStruct((N_NODES, 1), jnp.float32),
        jax.ShapeDtypeStruct((N_NODES, 1), jnp.float32),
        jax.ShapeDtypeStruct((N_NODES, 1), jnp.float32),
    ),
)


_pre_call = _build_pre()


def _post_body(hlin_ref, agga_ref, aggb_ref, n1_ref, n2_ref, ai_ref,
               war_ref, bar_ref, wfc_ref, bfc_ref, out_ref):
    agg1 = jnp.concatenate([agga_ref[0], aggb_ref[0]], axis=1)
    agg2 = jnp.concatenate([agga_ref[1], aggb_ref[1]], axis=1)
    h1 = agg1 * n1_ref[...]
    h2 = agg2 * n2_ref[...]
    war = war_ref[...]
    bar = bar_ref[0]
    ai = ai_ref[...]
    aj1 = jnp.sum(h1 * war, axis=1, keepdims=True) + bar
    aj2 = jnp.sum(h2 * war, axis=1, keepdims=True) + bar

    def act(x):
        return jnp.clip(jnp.exp(jnp.where(x >= 0, x, 0.2 * x)), -10.0, 10.0)

    a1 = act(ai + aj1)
    a2 = act(ai + aj2)
    inv = 1.0 / (a1 + a2)
    mix = (a1 * inv) * h1 + (a2 * inv) * h2
    out_ref[...] = (
        jnp.dot(mix, wfc_ref[...].T, preferred_element_type=jnp.float32)
        + bfc_ref[...][None, :]
    )


def _build_post(interpret=False):
    return pl.pallas_call(
        _post_body,
        interpret=interpret,
        grid=(ROW_GRID,),
        in_specs=[
        pl.BlockSpec((ROW_BLK, FEATS), lambda i: (i, 0)),
        pl.BlockSpec((2, ROW_BLK, HFEATS), lambda i: (0, i, 0)),
        pl.BlockSpec((2, ROW_BLK, HFEATS), lambda i: (0, i, 0)),
        pl.BlockSpec((ROW_BLK, 1), lambda i: (i, 0)),
        pl.BlockSpec((ROW_BLK, 1), lambda i: (i, 0)),
        pl.BlockSpec((ROW_BLK, 1), lambda i: (i, 0)),
        pl.BlockSpec((1, FEATS), lambda i: (0, 0)),
        pl.BlockSpec(memory_space=pltpu.SMEM),
        pl.BlockSpec((FEATS, FEATS), lambda i: (0, 0)),
        pl.BlockSpec((FEATS,), lambda i: (0,)),
    ],
        out_specs=pl.BlockSpec((ROW_BLK, FEATS), lambda i: (i, 0)),
        out_shape=jax.ShapeDtypeStruct((N_NODES, FEATS), jnp.float32),
    )


_post_call = _build_post()


def _edge_slabs(edge_index, src_offset):
    pad = EDGES_PER_TILE * NUM_TILES - edge_index.shape[1]
    src = jnp.concatenate(
        [edge_index[0] + src_offset, jnp.zeros((pad,), jnp.int32)]
    )
    dst = jnp.concatenate(
        [edge_index[1], jnp.full((pad,), N_NODES, jnp.int32)]
    )
    shape = (NUM_TILES * CHUNKS, LANES)
    return src.reshape(shape), dst.reshape(shape)


def kernel(h, edge_index1, edge_index2, W_lin, b_lin, W_fc, b_fc,
           W_al, b_al, W_ar, b_ar):
    src1, dst1 = _edge_slabs(edge_index1, 0)
    src2, dst2 = _edge_slabs(edge_index2, N_NODES)
    srcs = jnp.concatenate([src1, src2], axis=0)
    dsts = jnp.concatenate([dst1, dst2], axis=0)

    deg = _deg_call()(dsts).reshape(NUM_CORES, N_PAD, DEG_W)
    deg1 = deg[0, :N_NODES, :1]
    deg2 = deg[1, :N_NODES, :1]

    hlin, ga, gb, n1, n2, ai = _pre_call(h, W_lin, b_lin, W_al, b_al,
                                         deg1, deg2)

    agga, aggb = _prop_call()(srcs, dsts,
                              ga.reshape(2 * N_NODES, HFEATS),
                              gb.reshape(2 * N_NODES, HFEATS))
    agga = agga.reshape(NUM_CORES, N_PAD, HFEATS)
    aggb = aggb.reshape(NUM_CORES, N_PAD, HFEATS)

    return _post_call(hlin, agga, aggb, n1, n2, ai, W_ar, b_ar, W_fc, b_fc)
